# baseline (speedup 1.0000x reference)
import functools

import jax
import jax.numpy as jnp
from jax import lax
from jax.experimental import pallas as pl
from jax.experimental.pallas import tpu as pltpu
from jax.experimental.pallas import tpu_sc as plsc

N = 10000
N_PAD = 10240
IN_DIM = 128
HID = 32
OUT_DIM = 2
NC = 2
NS = 16
CHUNK = 128
NCHUNK0 = 116
NCHUNK1 = 44
NCHUNK_TOT = NS * (NCHUNK0 + NCHUNK1)
E_REAL = NCHUNK_TOT * CHUNK
NCHUNK_PAD = 2688
E_PAD = NCHUNK_PAD * CHUNK
NBUF = 4
ROWS_PER_TILE = N_PAD // NS

_F32 = jnp.float32


def _spmm_body(h_hbm, src_hbm, dst_hbm, out_hbm, src_v, dst_v, rows_v, zrow_v,
               acc, sem):
    c = lax.axis_index("c")
    s = lax.axis_index("s")
    base = jnp.where(c == 0, s * NCHUNK0, NS * NCHUNK0 + s * NCHUNK1)
    count = jnp.where(c == 0, NCHUNK0, NCHUNK1)
    pltpu.sync_copy(src_hbm.at[pl.ds(base, NCHUNK0)], src_v)
    pltpu.sync_copy(dst_hbm.at[pl.ds(base, NCHUNK0)], dst_v)
    zero16 = jnp.zeros((16,), _F32)

    def _z(i, carry):
        zrow_v[i, pl.ds(0, 16)] = zero16
        zrow_v[i, pl.ds(16, 16)] = zero16
        return carry

    lax.fori_loop(0, CHUNK, _z, 0)
    for k in range(ROWS_PER_TILE // CHUNK):
        pltpu.sync_copy(zrow_v, acc.at[pl.ds(s * ROWS_PER_TILE + k * CHUNK, CHUNK)])
    plsc.subcore_barrier()

    for b in range(NBUF):
        pltpu.async_copy(h_hbm.at[src_v.at[b]], rows_v.at[b], sem[b])

    def _grp(g, carry):
        for b in range(NBUF):
            j = g * NBUF + b
            pltpu.make_async_copy(h_hbm.at[src_v.at[0]], rows_v.at[b],
                                  sem[b]).wait()
            pltpu.sync_copy(rows_v.at[b], acc.at[dst_v.at[j]], add=True)

            @pl.when(j + NBUF < count)
            def _():
                pltpu.async_copy(h_hbm.at[src_v.at[j + NBUF]], rows_v.at[b],
                                 sem[b])
        return carry

    lax.fori_loop(0, count // NBUF, _grp, 0)
    plsc.subcore_barrier()
    pltpu.sync_copy(acc.at[pl.ds(s * ROWS_PER_TILE, ROWS_PER_TILE)],
                    out_hbm.at[c, pl.ds(s * ROWS_PER_TILE, ROWS_PER_TILE)])


def _deg_body(dst_hbm, out_hbm, dst_v, ones_v, zbuf_v, acc, sem):
    c = lax.axis_index("c")
    s = lax.axis_index("s")
    base = jnp.where(c == 0, s * NCHUNK0, NS * NCHUNK0 + s * NCHUNK1)
    count = jnp.where(c == 0, NCHUNK0, NCHUNK1)
    pltpu.sync_copy(dst_hbm.at[pl.ds(base, NCHUNK0)], dst_v)
    one16 = jnp.full((16,), 1.0, _F32)
    for i in range(CHUNK // 16):
        ones_v[pl.ds(i * 16, 16)] = one16
    zero16 = jnp.zeros((16,), _F32)

    def _z(i, carry):
        zbuf_v[pl.ds(i * 16, 16)] = zero16
        return carry

    lax.fori_loop(0, ROWS_PER_TILE // 16, _z, 0)
    pltpu.sync_copy(zbuf_v, acc.at[pl.ds(s * ROWS_PER_TILE, ROWS_PER_TILE)])
    plsc.subcore_barrier()

    def _edge_chunk(j, carry):
        pltpu.sync_copy(ones_v, acc.at[dst_v.at[j]], add=True)
        return carry

    lax.fori_loop(0, count, _edge_chunk, 0)
    plsc.subcore_barrier()
    pltpu.sync_copy(acc.at[pl.ds(s * ROWS_PER_TILE, ROWS_PER_TILE)],
                    out_hbm.at[c, pl.ds(s * ROWS_PER_TILE, ROWS_PER_TILE)])


def _make_spmm():
    mesh = plsc.VectorSubcoreMesh(core_axis_name="c", subcore_axis_name="s")
    return pl.kernel(
        _spmm_body,
        out_type=jax.ShapeDtypeStruct((NC, N_PAD, HID), _F32),
        mesh=mesh,
        scratch_types=[
            pltpu.VMEM((NCHUNK0, CHUNK), jnp.int32),
            pltpu.VMEM((NCHUNK0, CHUNK), jnp.int32),
            pltpu.VMEM((NBUF, CHUNK, HID), _F32),
            pltpu.VMEM((CHUNK, HID), _F32),
            pltpu.VMEM_SHARED((N_PAD, HID), _F32),
            [pltpu.SemaphoreType.DMA] * NBUF,
        ],
        compiler_params=pltpu.CompilerParams(use_tc_tiling_on_sc=False),
        name="gcn_spmm_sc",
    )


def _make_deg():
    mesh = plsc.VectorSubcoreMesh(core_axis_name="c", subcore_axis_name="s")
    return pl.kernel(
        _deg_body,
        out_type=jax.ShapeDtypeStruct((NC, N_PAD), _F32),
        mesh=mesh,
        scratch_types=[
            pltpu.VMEM((NCHUNK0, CHUNK), jnp.int32),
            pltpu.VMEM((CHUNK,), _F32),
            pltpu.VMEM((ROWS_PER_TILE,), _F32),
            pltpu.VMEM_SHARED((N_PAD,), _F32),
            pltpu.SemaphoreType.DMA,
        ],
        compiler_params=pltpu.CompilerParams(use_tc_tiling_on_sc=False),
        name="gcn_deg_sc",
    )


_BLK = 1280


def _tc0_body(deg_ref, x_ref, w_ref, dinv_ref, h_ref):
    deg = deg_ref[:, 0:1] + deg_ref[:, 1:2] + 1.0
    dinv = lax.rsqrt(deg)
    dinv32 = jnp.broadcast_to(dinv, (dinv.shape[0], HID))
    dinv_ref[...] = dinv32
    h_ref[...] = dinv32 * jnp.dot(x_ref[...], w_ref[...],
                                  preferred_element_type=_F32)


def _tc_mid_body(agg_ref, h_ref, dinv_ref, w_ref, b_ref, out_ref):
    a = agg_ref[0] + agg_ref[1] + h_ref[...]
    xn = jnp.maximum(dinv_ref[...] * a + b_ref[...], 0.0)
    out_ref[...] = dinv_ref[...] * jnp.dot(xn, w_ref[...],
                                           preferred_element_type=_F32)


def _tc_last_body(agg_ref, h_ref, dinv_ref, w_ref, b_ref, out_ref):
    a = dinv_ref[...] * (agg_ref[0] + agg_ref[1] + h_ref[...])
    out_ref[...] = jnp.dot(a, w_ref[...], preferred_element_type=_F32) + b_ref[...]


def _tc0(degT, x_pad, W1):
    grid = (N_PAD // _BLK,)
    return pl.pallas_call(
        _tc0_body,
        grid=grid,
        in_specs=[
            pl.BlockSpec((_BLK, NC), lambda i: (i, 0)),
            pl.BlockSpec((_BLK, IN_DIM), lambda i: (i, 0)),
            pl.BlockSpec((IN_DIM, HID), lambda i: (0, 0)),
        ],
        out_specs=[
            pl.BlockSpec((_BLK, HID), lambda i: (i, 0)),
            pl.BlockSpec((_BLK, HID), lambda i: (i, 0)),
        ],
        out_shape=[
            jax.ShapeDtypeStruct((N_PAD, HID), _F32),
            jax.ShapeDtypeStruct((N_PAD, HID), _F32),
        ],
        name="gcn_tc0",
    )(degT, x_pad, W1)


def _tc_mid(agg, h, dinv32, W, b):
    grid = (N_PAD // _BLK,)
    return pl.pallas_call(
        _tc_mid_body,
        grid=grid,
        in_specs=[
            pl.BlockSpec((NC, _BLK, HID), lambda i: (0, i, 0)),
            pl.BlockSpec((_BLK, HID), lambda i: (i, 0)),
            pl.BlockSpec((_BLK, HID), lambda i: (i, 0)),
            pl.BlockSpec((HID, HID), lambda i: (0, 0)),
            pl.BlockSpec((1, HID), lambda i: (0, 0)),
        ],
        out_specs=pl.BlockSpec((_BLK, HID), lambda i: (i, 0)),
        out_shape=jax.ShapeDtypeStruct((N_PAD, HID), _F32),
        name="gcn_tc_mid",
    )(agg, h, dinv32, W, b)


def _tc_last(agg, h, dinv32, W4, b4):
    grid = (N_PAD // _BLK,)
    return pl.pallas_call(
        _tc_last_body,
        grid=grid,
        in_specs=[
            pl.BlockSpec((NC, _BLK, HID), lambda i: (0, i, 0)),
            pl.BlockSpec((_BLK, HID), lambda i: (i, 0)),
            pl.BlockSpec((_BLK, HID), lambda i: (i, 0)),
            pl.BlockSpec((HID, OUT_DIM), lambda i: (0, 0)),
            pl.BlockSpec((1, OUT_DIM), lambda i: (0, 0)),
        ],
        out_specs=pl.BlockSpec((_BLK, OUT_DIM), lambda i: (i, 0)),
        out_shape=jax.ShapeDtypeStruct((N_PAD, OUT_DIM), _F32),
        name="gcn_tc_last",
    )(agg, h, dinv32, W4, b4)


def kernel(x, edge_index, W1, b1, W2, b2, W3, b3, W4, b4):
    src = edge_index[0].astype(jnp.int32)
    dst = edge_index[1].astype(jnp.int32)
    pad = E_PAD - src.shape[0]
    src_p = jnp.concatenate([src, jnp.zeros((pad,), jnp.int32)])
    dst_p = jnp.concatenate([dst, jnp.full((pad,), N, jnp.int32)])
    src_p = src_p.reshape(NCHUNK_PAD, CHUNK)
    dst_p = dst_p.reshape(NCHUNK_PAD, CHUNK)
    x_pad = jnp.pad(x, ((0, N_PAD - N), (0, 0)))

    spmm = _make_spmm()
    degp = _make_deg()(dst_p)
    degT = degp.T

    dinv32, h1 = _tc0(degT, x_pad, W1)
    agg1 = spmm(h1, src_p, dst_p)
    h2 = _tc_mid(agg1, h1, dinv32, W2, b1.reshape(1, HID))
    agg2 = spmm(h2, src_p, dst_p)
    h3 = _tc_mid(agg2, h2, dinv32, W3, b2.reshape(1, HID))
    agg3 = spmm(h3, src_p, dst_p)
    x4 = _tc_mid(agg3, h3, dinv32, jnp.eye(HID, dtype=_F32),
                 b3.reshape(1, HID))
    agg4 = spmm(x4, src_p, dst_p)
    out = _tc_last(agg4, x4, dinv32, W4, b4.reshape(1, OUT_DIM))
    return out[:N]


# speedup vs baseline: 2.1441x; 2.1441x over previous
import functools

import jax
import jax.numpy as jnp
from jax import lax
from jax.experimental import pallas as pl
from jax.experimental.pallas import tpu as pltpu
from jax.experimental.pallas import tpu_sc as plsc

N = 10000
N_PAD = 10240
IN_DIM = 128
HID = 32
OUT_DIM = 2
NC = 2
NS = 16
CHUNK = 128
NCHUNK0 = 80
NCHUNK1 = 80
NCHUNK_PAD = NC * NS * NCHUNK0
E_PAD = NCHUNK_PAD * CHUNK
NBUF = 4
ROWS_PER_TILE = N_PAD // NS

_F32 = jnp.float32


def _spmm_body(h_hbm, src_hbm, dst_hbm, out_hbm, src_v, dst_v, rows_v, zrow_v,
               acc, sem):
    c = lax.axis_index("c")
    s = lax.axis_index("s")
    base = (c * NS + s) * NCHUNK0
    count = NCHUNK0
    pltpu.sync_copy(src_hbm.at[pl.ds(base, NCHUNK0)], src_v)
    pltpu.sync_copy(dst_hbm.at[pl.ds(base, NCHUNK0)], dst_v)
    zero16 = jnp.zeros((16,), _F32)

    def _z(i, carry):
        zrow_v[i, pl.ds(0, 16)] = zero16
        zrow_v[i, pl.ds(16, 16)] = zero16
        return carry

    lax.fori_loop(0, CHUNK, _z, 0)
    for k in range(ROWS_PER_TILE // CHUNK):
        pltpu.sync_copy(zrow_v, acc.at[pl.ds(s * ROWS_PER_TILE + k * CHUNK, CHUNK)])
    plsc.subcore_barrier()

    for b in range(NBUF):
        pltpu.async_copy(h_hbm.at[src_v.at[b]], rows_v.at[b], sem[b])

    def _grp(g, carry):
        for b in range(NBUF):
            j = g * NBUF + b
            pltpu.make_async_copy(h_hbm.at[src_v.at[0]], rows_v.at[b],
                                  sem[b]).wait()
            pltpu.sync_copy(rows_v.at[b], acc.at[dst_v.at[j]], add=True)

            @pl.when(j + NBUF < count)
            def _():
                pltpu.async_copy(h_hbm.at[src_v.at[j + NBUF]], rows_v.at[b],
                                 sem[b])
        return carry

    lax.fori_loop(0, count // NBUF, _grp, 0)
    plsc.subcore_barrier()
    pltpu.sync_copy(acc.at[pl.ds(s * ROWS_PER_TILE, ROWS_PER_TILE)],
                    out_hbm.at[c, pl.ds(s * ROWS_PER_TILE, ROWS_PER_TILE)])


def _deg_body(dst_hbm, out_hbm, dst_v, ones_v, zbuf_v, acc, sem):
    c = lax.axis_index("c")
    s = lax.axis_index("s")
    base = (c * NS + s) * NCHUNK0
    count = NCHUNK0
    pltpu.sync_copy(dst_hbm.at[pl.ds(base, NCHUNK0)], dst_v)
    one16 = jnp.full((16,), 1.0, _F32)
    for i in range(CHUNK // 16):
        ones_v[pl.ds(i * 16, 16)] = one16
    zero16 = jnp.zeros((16,), _F32)

    def _z(i, carry):
        zbuf_v[pl.ds(i * 16, 16)] = zero16
        return carry

    lax.fori_loop(0, ROWS_PER_TILE // 16, _z, 0)
    pltpu.sync_copy(zbuf_v, acc.at[pl.ds(s * ROWS_PER_TILE, ROWS_PER_TILE)])
    plsc.subcore_barrier()

    def _edge_chunk(j, carry):
        pltpu.sync_copy(ones_v, acc.at[dst_v.at[j]], add=True)
        return carry

    lax.fori_loop(0, count, _edge_chunk, 0)
    plsc.subcore_barrier()
    pltpu.sync_copy(acc.at[pl.ds(s * ROWS_PER_TILE, ROWS_PER_TILE)],
                    out_hbm.at[c, pl.ds(s * ROWS_PER_TILE, ROWS_PER_TILE)])


def _make_spmm():
    mesh = plsc.VectorSubcoreMesh(core_axis_name="c", subcore_axis_name="s")
    return pl.kernel(
        _spmm_body,
        out_type=jax.ShapeDtypeStruct((NC, N_PAD, HID), _F32),
        mesh=mesh,
        scratch_types=[
            pltpu.VMEM((NCHUNK0, CHUNK), jnp.int32),
            pltpu.VMEM((NCHUNK0, CHUNK), jnp.int32),
            pltpu.VMEM((NBUF, CHUNK, HID), _F32),
            pltpu.VMEM((CHUNK, HID), _F32),
            pltpu.VMEM_SHARED((N_PAD, HID), _F32),
            [pltpu.SemaphoreType.DMA] * NBUF,
        ],
        compiler_params=pltpu.CompilerParams(use_tc_tiling_on_sc=False),
        name="gcn_spmm_sc",
    )


def _make_deg():
    mesh = plsc.VectorSubcoreMesh(core_axis_name="c", subcore_axis_name="s")
    return pl.kernel(
        _deg_body,
        out_type=jax.ShapeDtypeStruct((NC, N_PAD), _F32),
        mesh=mesh,
        scratch_types=[
            pltpu.VMEM((NCHUNK0, CHUNK), jnp.int32),
            pltpu.VMEM((CHUNK,), _F32),
            pltpu.VMEM((ROWS_PER_TILE,), _F32),
            pltpu.VMEM_SHARED((N_PAD,), _F32),
            pltpu.SemaphoreType.DMA,
        ],
        compiler_params=pltpu.CompilerParams(use_tc_tiling_on_sc=False),
        name="gcn_deg_sc",
    )


_BLK = 1280


def _tc0_body(deg_ref, x_ref, w_ref, dinv_ref, h_ref):
    deg = deg_ref[:, 0:1] + deg_ref[:, 1:2] + 1.0
    dinv = lax.rsqrt(deg)
    dinv32 = jnp.broadcast_to(dinv, (dinv.shape[0], HID))
    dinv_ref[...] = dinv32
    h_ref[...] = dinv32 * jnp.dot(x_ref[...], w_ref[...],
                                  preferred_element_type=_F32)


def _tc_mid_body(agg_ref, h_ref, dinv_ref, w_ref, b_ref, out_ref):
    a = agg_ref[0] + agg_ref[1] + h_ref[...]
    xn = jnp.maximum(dinv_ref[...] * a + b_ref[...], 0.0)
    out_ref[...] = dinv_ref[...] * jnp.dot(xn, w_ref[...],
                                           preferred_element_type=_F32)


def _tc_last_body(agg_ref, h_ref, dinv_ref, w_ref, b_ref, out_ref):
    a = dinv_ref[...] * (agg_ref[0] + agg_ref[1] + h_ref[...])
    out_ref[...] = jnp.dot(a, w_ref[...], preferred_element_type=_F32) + b_ref[...]


def _tc0(degT, x_pad, W1):
    grid = (N_PAD // _BLK,)
    return pl.pallas_call(
        _tc0_body,
        grid=grid,
        in_specs=[
            pl.BlockSpec((_BLK, NC), lambda i: (i, 0)),
            pl.BlockSpec((_BLK, IN_DIM), lambda i: (i, 0)),
            pl.BlockSpec((IN_DIM, HID), lambda i: (0, 0)),
        ],
        out_specs=[
            pl.BlockSpec((_BLK, HID), lambda i: (i, 0)),
            pl.BlockSpec((_BLK, HID), lambda i: (i, 0)),
        ],
        out_shape=[
            jax.ShapeDtypeStruct((N_PAD, HID), _F32),
            jax.ShapeDtypeStruct((N_PAD, HID), _F32),
        ],
        name="gcn_tc0",
    )(degT, x_pad, W1)


def _tc_mid(agg, h, dinv32, W, b):
    grid = (N_PAD // _BLK,)
    return pl.pallas_call(
        _tc_mid_body,
        grid=grid,
        in_specs=[
            pl.BlockSpec((NC, _BLK, HID), lambda i: (0, i, 0)),
            pl.BlockSpec((_BLK, HID), lambda i: (i, 0)),
            pl.BlockSpec((_BLK, HID), lambda i: (i, 0)),
            pl.BlockSpec((HID, HID), lambda i: (0, 0)),
            pl.BlockSpec((1, HID), lambda i: (0, 0)),
        ],
        out_specs=pl.BlockSpec((_BLK, HID), lambda i: (i, 0)),
        out_shape=jax.ShapeDtypeStruct((N_PAD, HID), _F32),
        name="gcn_tc_mid",
    )(agg, h, dinv32, W, b)


def _tc_last(agg, h, dinv32, W4, b4):
    grid = (N_PAD // _BLK,)
    return pl.pallas_call(
        _tc_last_body,
        grid=grid,
        in_specs=[
            pl.BlockSpec((NC, _BLK, HID), lambda i: (0, i, 0)),
            pl.BlockSpec((_BLK, HID), lambda i: (i, 0)),
            pl.BlockSpec((_BLK, HID), lambda i: (i, 0)),
            pl.BlockSpec((HID, OUT_DIM), lambda i: (0, 0)),
            pl.BlockSpec((1, OUT_DIM), lambda i: (0, 0)),
        ],
        out_specs=pl.BlockSpec((_BLK, OUT_DIM), lambda i: (i, 0)),
        out_shape=jax.ShapeDtypeStruct((N_PAD, OUT_DIM), _F32),
        name="gcn_tc_last",
    )(agg, h, dinv32, W4, b4)


def kernel(x, edge_index, W1, b1, W2, b2, W3, b3, W4, b4):
    src = edge_index[0].astype(jnp.int32)
    dst = edge_index[1].astype(jnp.int32)
    pad = E_PAD - src.shape[0]
    seq = jnp.arange(pad, dtype=jnp.int32)
    src_p = jnp.concatenate([src, seq % N])
    dst_p = jnp.concatenate([dst, N + seq % (N_PAD - N)])
    src_p = src_p.reshape(NCHUNK_PAD, CHUNK)
    dst_p = dst_p.reshape(NCHUNK_PAD, CHUNK)
    x_pad = jnp.pad(x, ((0, N_PAD - N), (0, 0)))

    spmm = _make_spmm()
    degp = _make_deg()(dst_p)
    degT = degp.T

    dinv32, h1 = _tc0(degT, x_pad, W1)
    agg1 = spmm(h1, src_p, dst_p)
    h2 = _tc_mid(agg1, h1, dinv32, W2, b1.reshape(1, HID))
    agg2 = spmm(h2, src_p, dst_p)
    h3 = _tc_mid(agg2, h2, dinv32, W3, b2.reshape(1, HID))
    agg3 = spmm(h3, src_p, dst_p)
    x4 = _tc_mid(agg3, h3, dinv32, jnp.eye(HID, dtype=_F32),
                 b3.reshape(1, HID))
    agg4 = spmm(x4, src_p, dst_p)
    out = _tc_last(agg4, x4, dinv32, W4, b4.reshape(1, OUT_DIM))
    return out[:N]


# speedup vs baseline: 2.9246x; 1.3640x over previous
import functools

import jax
import jax.numpy as jnp
import numpy as np
from jax import lax
from jax.experimental import pallas as pl
from jax.experimental.pallas import tpu as pltpu
from jax.experimental.pallas import tpu_sc as plsc

N = 10000
N_PAD = 10240
IN_DIM = 128
HID = 32
OUT_DIM = 2
NC = 2
NS = 16
CHUNK = 128
NCHUNK = 80
NBUF = 4
E = 320000
NCHUNK_REAL = E // CHUNK
NCHUNK_TOT = NC * NS * NCHUNK
NPADCH = NCHUNK_TOT - NCHUNK_REAL
ROWS_PER_TILE = N_PAD // NS
NFLAT = N_PAD * HID // 128

_F32 = jnp.float32


def _spmm_body(h_hbm, src_hbm, dst_hbm, psrc_hbm, pdst_hbm, out_hbm,
               src_v, dst_v, rows_v, zrow_v, acc, sem):
    c = lax.axis_index("c")
    s = lax.axis_index("s")
    w = c * NS + s
    base = w * NCHUNK
    @pl.when(w < NC * NS - 1)
    def _():
        pltpu.sync_copy(src_hbm.at[pl.ds(base, NCHUNK)], src_v)
        pltpu.sync_copy(dst_hbm.at[pl.ds(base, NCHUNK)], dst_v)

    @pl.when(w == NC * NS - 1)
    def _():
        nreal = NCHUNK - NPADCH
        pltpu.sync_copy(src_hbm.at[pl.ds(NCHUNK_REAL - nreal, nreal)],
                        src_v.at[pl.ds(0, nreal)])
        pltpu.sync_copy(dst_hbm.at[pl.ds(NCHUNK_REAL - nreal, nreal)],
                        dst_v.at[pl.ds(0, nreal)])
        pltpu.sync_copy(psrc_hbm, src_v.at[pl.ds(nreal, NPADCH)])
        pltpu.sync_copy(pdst_hbm, dst_v.at[pl.ds(nreal, NPADCH)])

    zero16 = jnp.zeros((16,), _F32)

    def _z(i, carry):
        zrow_v[i, pl.ds(0, 16)] = zero16
        zrow_v[i, pl.ds(16, 16)] = zero16
        return carry

    lax.fori_loop(0, CHUNK, _z, 0)
    for k in range(ROWS_PER_TILE // CHUNK):
        pltpu.sync_copy(zrow_v, acc.at[pl.ds(s * ROWS_PER_TILE + k * CHUNK, CHUNK)])
    plsc.subcore_barrier()

    for b in range(NBUF):
        pltpu.async_copy(h_hbm.at[src_v.at[b]], rows_v.at[b], sem[b])

    def _grp(g, carry):
        for b in range(NBUF):
            j = g * NBUF + b
            pltpu.make_async_copy(h_hbm.at[src_v.at[0]], rows_v.at[b],
                                  sem[b]).wait()
            pltpu.sync_copy(rows_v.at[b], acc.at[dst_v.at[j]], add=True)

            @pl.when(j + NBUF < NCHUNK)
            def _():
                pltpu.async_copy(h_hbm.at[src_v.at[j + NBUF]], rows_v.at[b],
                                 sem[b])
        return carry

    lax.fori_loop(0, NCHUNK // NBUF, _grp, 0)
    plsc.subcore_barrier()
    pltpu.sync_copy(acc.at[pl.ds(s * ROWS_PER_TILE, ROWS_PER_TILE)],
                    out_hbm.at[c, pl.ds(s * ROWS_PER_TILE, ROWS_PER_TILE)])


def _deg_body(dst_hbm, pdst_hbm, out_hbm, dst_v, ones_v, zbuf_v, acc, sem):
    c = lax.axis_index("c")
    s = lax.axis_index("s")
    w = c * NS + s
    base = w * NCHUNK

    @pl.when(w < NC * NS - 1)
    def _():
        pltpu.sync_copy(dst_hbm.at[pl.ds(base, NCHUNK)], dst_v)

    @pl.when(w == NC * NS - 1)
    def _():
        nreal = NCHUNK - NPADCH
        pltpu.sync_copy(dst_hbm.at[pl.ds(NCHUNK_REAL - nreal, nreal)],
                        dst_v.at[pl.ds(0, nreal)])
        pltpu.sync_copy(pdst_hbm, dst_v.at[pl.ds(nreal, NPADCH)])

    one16 = jnp.full((16,), 1.0, _F32)
    for i in range(CHUNK // 16):
        ones_v[pl.ds(i * 16, 16)] = one16
    zero16 = jnp.zeros((16,), _F32)

    def _z(i, carry):
        zbuf_v[pl.ds(i * 16, 16)] = zero16
        return carry

    lax.fori_loop(0, ROWS_PER_TILE // 16, _z, 0)
    pltpu.sync_copy(zbuf_v, acc.at[pl.ds(s * ROWS_PER_TILE, ROWS_PER_TILE)])
    plsc.subcore_barrier()

    def _edge_chunk(j, carry):
        pltpu.sync_copy(ones_v, acc.at[dst_v.at[j]], add=True)
        return carry

    lax.fori_loop(0, NCHUNK, _edge_chunk, 0)
    plsc.subcore_barrier()
    pltpu.sync_copy(acc.at[pl.ds(s * ROWS_PER_TILE, ROWS_PER_TILE)],
                    out_hbm.at[c, pl.ds(s * ROWS_PER_TILE, ROWS_PER_TILE)])


def _make_spmm():
    mesh = plsc.VectorSubcoreMesh(core_axis_name="c", subcore_axis_name="s")
    return pl.kernel(
        _spmm_body,
        out_type=jax.ShapeDtypeStruct((NC, N_PAD, HID), _F32),
        mesh=mesh,
        scratch_types=[
            pltpu.VMEM((NCHUNK, CHUNK), jnp.int32),
            pltpu.VMEM((NCHUNK, CHUNK), jnp.int32),
            pltpu.VMEM((NBUF, CHUNK, HID), _F32),
            pltpu.VMEM((CHUNK, HID), _F32),
            pltpu.VMEM_SHARED((N_PAD, HID), _F32),
            [pltpu.SemaphoreType.DMA] * NBUF,
        ],
        compiler_params=pltpu.CompilerParams(use_tc_tiling_on_sc=False),
        name="gcn_spmm_sc",
    )


def _make_deg():
    mesh = plsc.VectorSubcoreMesh(core_axis_name="c", subcore_axis_name="s")
    return pl.kernel(
        _deg_body,
        out_type=jax.ShapeDtypeStruct((NC, N_PAD), _F32),
        mesh=mesh,
        scratch_types=[
            pltpu.VMEM((NCHUNK, CHUNK), jnp.int32),
            pltpu.VMEM((CHUNK,), _F32),
            pltpu.VMEM((ROWS_PER_TILE,), _F32),
            pltpu.VMEM_SHARED((N_PAD,), _F32),
            pltpu.SemaphoreType.DMA,
        ],
        compiler_params=pltpu.CompilerParams(use_tc_tiling_on_sc=False),
        name="gcn_deg_sc",
    )


_BLK = 640


def _tc0_body(x_ref, w_ref, dinv_ref, h_ref):
    h_ref[...] = dinv_ref[...] * jnp.dot(x_ref[...], w_ref[...],
                                         preferred_element_type=_F32)


def _tc_mid_body(agg_ref, h_ref, dinv_ref, w_ref, b_ref, out_ref):
    a = agg_ref[0] + agg_ref[1] + h_ref[...]
    xn = jnp.maximum(dinv_ref[...] * a + b_ref[...], 0.0)
    out_ref[...] = dinv_ref[...] * jnp.dot(xn, w_ref[...],
                                           preferred_element_type=_F32)


def _tc_mid_nomm_body(agg_ref, h_ref, dinv_ref, b_ref, out_ref):
    a = agg_ref[0] + agg_ref[1] + h_ref[...]
    out_ref[...] = dinv_ref[...] * jnp.maximum(
        dinv_ref[...] * a + b_ref[...], 0.0)


def _tc_last_body(agg_ref, h_ref, dinv_ref, w_ref, b_ref, out_ref):
    a = dinv_ref[...] * (agg_ref[0] + agg_ref[1] + h_ref[...])
    out_ref[...] = jnp.dot(a, w_ref[...], preferred_element_type=_F32) + b_ref[...]


def _row_spec(width=128):
    return pl.BlockSpec((_BLK, width), lambda i: (i, 0))


def _full_spec(shape):
    nd = len(shape)
    return pl.BlockSpec(shape, lambda i: (0,) * nd)


def _tc0(x4, W1bd, dinv_rep):
    return pl.pallas_call(
        _tc0_body,
        grid=(NFLAT // _BLK,),
        in_specs=[
            pl.BlockSpec((_BLK, 4 * IN_DIM), lambda i: (i, 0)),
            _full_spec((4 * IN_DIM, 128)),
            _row_spec(),
        ],
        out_specs=_row_spec(),
        out_shape=jax.ShapeDtypeStruct((NFLAT, 128), _F32),
        name="gcn_tc0",
    )(x4, W1bd, dinv_rep)


def _tc_mid(agg, h, dinv_rep, Wbd, b_rep):
    return pl.pallas_call(
        _tc_mid_body,
        grid=(NFLAT // _BLK,),
        in_specs=[
            pl.BlockSpec((NC, _BLK, 128), lambda i: (0, i, 0)),
            _row_spec(), _row_spec(),
            _full_spec((128, 128)),
            _full_spec((1, 128)),
        ],
        out_specs=_row_spec(),
        out_shape=jax.ShapeDtypeStruct((NFLAT, 128), _F32),
        name="gcn_tc_mid",
    )(agg, h, dinv_rep, Wbd, b_rep)


def _tc_mid_nomm(agg, h, dinv_rep, b_rep):
    return pl.pallas_call(
        _tc_mid_nomm_body,
        grid=(NFLAT // _BLK,),
        in_specs=[
            pl.BlockSpec((NC, _BLK, 128), lambda i: (0, i, 0)),
            _row_spec(), _row_spec(),
            _full_spec((1, 128)),
        ],
        out_specs=_row_spec(),
        out_shape=jax.ShapeDtypeStruct((NFLAT, 128), _F32),
        name="gcn_tc_mid2",
    )(agg, h, dinv_rep, b_rep)


def _tc_last(agg, h, dinv_rep, W4bd, b4_rep):
    return pl.pallas_call(
        _tc_last_body,
        grid=(NFLAT // _BLK,),
        in_specs=[
            pl.BlockSpec((NC, _BLK, 128), lambda i: (0, i, 0)),
            _row_spec(), _row_spec(),
            _full_spec((128, 4 * OUT_DIM)),
            _full_spec((1, 4 * OUT_DIM)),
        ],
        out_specs=pl.BlockSpec((_BLK, 4 * OUT_DIM), lambda i: (i, 0)),
        out_shape=jax.ShapeDtypeStruct((NFLAT, 4 * OUT_DIM), _F32),
        name="gcn_tc_last",
    )(agg, h, dinv_rep, W4bd, b4_rep)


def _kron4(W):
    return jnp.kron(jnp.eye(4, dtype=_F32), W)


def kernel(x, edge_index, W1, b1, W2, b2, W3, b3, W4, b4):
    src2d = edge_index[0].astype(jnp.int32).reshape(NCHUNK_REAL, CHUNK)
    dst2d = edge_index[1].astype(jnp.int32).reshape(NCHUNK_REAL, CHUNK)
    seq = jnp.arange(NPADCH * CHUNK, dtype=jnp.int32)
    psrc = (seq % N).reshape(NPADCH, CHUNK)
    pdst = (N + seq % (N_PAD - N)).reshape(NPADCH, CHUNK)

    x4 = jnp.pad(x, ((0, N_PAD - N), (0, 0))).reshape(NFLAT, 4 * IN_DIM)

    spmm = _make_spmm()
    degp = _make_deg()(dst2d, pdst)
    dinv = lax.rsqrt(degp[0] + degp[1] + 1.0)
    dinv_rep = jnp.repeat(dinv, HID).reshape(NFLAT, 128)

    def flat(a):
        return a.reshape(NC, NFLAT, 128)

    h1 = _tc0(x4, _kron4(W1), dinv_rep)
    agg1 = flat(spmm(h1.reshape(N_PAD, HID), src2d, dst2d, psrc, pdst))
    h2 = _tc_mid(agg1, h1, dinv_rep, _kron4(W2), jnp.tile(b1, 4)[None, :])
    agg2 = flat(spmm(h2.reshape(N_PAD, HID), src2d, dst2d, psrc, pdst))
    h3 = _tc_mid(agg2, h2, dinv_rep, _kron4(W3), jnp.tile(b2, 4)[None, :])
    agg3 = flat(spmm(h3.reshape(N_PAD, HID), src2d, dst2d, psrc, pdst))
    x4p = _tc_mid_nomm(agg3, h3, dinv_rep, jnp.tile(b3, 4)[None, :])
    agg4 = flat(spmm(x4p.reshape(N_PAD, HID), src2d, dst2d, psrc, pdst))
    out = _tc_last(agg4, x4p, dinv_rep, _kron4(W4), jnp.tile(b4, 4)[None, :])
    return out.reshape(N_PAD, OUT_DIM)[:N]


# speedup vs baseline: 3.4025x; 1.1634x over previous
import functools

import jax
import jax.numpy as jnp
import numpy as np
from jax import lax
from jax.experimental import pallas as pl
from jax.experimental.pallas import tpu as pltpu
from jax.experimental.pallas import tpu_sc as plsc

N = 10000
N_PAD = 10240
IN_DIM = 128
HID = 32
OUT_DIM = 2
NC = 2
NS = 16
CHUNK = 128
NCHUNK = 80
NBUF = 8
E = 320000
NCHUNK_REAL = E // CHUNK
NCHUNK_TOT = NC * NS * NCHUNK
NPADCH = NCHUNK_TOT - NCHUNK_REAL
ROWS_PER_TILE = N_PAD // NS
NFLAT = N_PAD * HID // 128

_F32 = jnp.float32


def _spmm_body(h_hbm, ed_hbm, ped_hbm, out_hbm,
               ed_v, rows_v, zrow_v, acc, sem):
    c = lax.axis_index("c")
    s = lax.axis_index("s")
    w = c * NS + s
    base = w * NCHUNK
    @pl.when(w < NC * NS - 1)
    def _():
        pltpu.sync_copy(ed_hbm.at[pl.ds(base, NCHUNK)], ed_v)

    @pl.when(w == NC * NS - 1)
    def _():
        nreal = NCHUNK - NPADCH
        pltpu.sync_copy(ed_hbm.at[pl.ds(NCHUNK_REAL - nreal, nreal)],
                        ed_v.at[pl.ds(0, nreal)])
        pltpu.sync_copy(ped_hbm, ed_v.at[pl.ds(nreal, NPADCH)])

    zero16 = jnp.zeros((16,), _F32)

    def _z(i, carry):
        zrow_v[i, pl.ds(0, 16)] = zero16
        zrow_v[i, pl.ds(16, 16)] = zero16
        return carry

    lax.fori_loop(0, CHUNK, _z, 0)
    for k in range(ROWS_PER_TILE // CHUNK):
        pltpu.sync_copy(zrow_v, acc.at[pl.ds(s * ROWS_PER_TILE + k * CHUNK, CHUNK)])
    plsc.subcore_barrier()

    for b in range(NBUF):
        pltpu.async_copy(h_hbm.at[ed_v.at[b, 0]], rows_v.at[b], sem[b])

    def _grp(g, carry):
        for b in range(NBUF):
            j = g * NBUF + b
            pltpu.make_async_copy(h_hbm.at[ed_v.at[0, 0]], rows_v.at[b],
                                  sem[b]).wait()
            pltpu.sync_copy(rows_v.at[b], acc.at[ed_v.at[j, 1]], add=True)

            @pl.when(j + NBUF < NCHUNK)
            def _():
                pltpu.async_copy(h_hbm.at[ed_v.at[j + NBUF, 0]], rows_v.at[b],
                                 sem[b])
        return carry

    lax.fori_loop(0, NCHUNK // NBUF, _grp, 0)
    plsc.subcore_barrier()
    pltpu.sync_copy(acc.at[pl.ds(s * ROWS_PER_TILE, ROWS_PER_TILE)],
                    out_hbm.at[c, pl.ds(s * ROWS_PER_TILE, ROWS_PER_TILE)])


def _deg_body(ed_hbm, ped_hbm, out_hbm, ed_v, ones_v, zbuf_v, acc, sem):
    c = lax.axis_index("c")
    s = lax.axis_index("s")
    w = c * NS + s
    base = w * NCHUNK

    @pl.when(w < NC * NS - 1)
    def _():
        pltpu.sync_copy(ed_hbm.at[pl.ds(base, NCHUNK)], ed_v)

    @pl.when(w == NC * NS - 1)
    def _():
        nreal = NCHUNK - NPADCH
        pltpu.sync_copy(ed_hbm.at[pl.ds(NCHUNK_REAL - nreal, nreal)],
                        ed_v.at[pl.ds(0, nreal)])
        pltpu.sync_copy(ped_hbm, ed_v.at[pl.ds(nreal, NPADCH)])

    one16 = jnp.full((16,), 1.0, _F32)
    for i in range(CHUNK // 16):
        ones_v[pl.ds(i * 16, 16)] = one16
    zero16 = jnp.zeros((16,), _F32)

    def _z(i, carry):
        zbuf_v[pl.ds(i * 16, 16)] = zero16
        return carry

    lax.fori_loop(0, ROWS_PER_TILE // 16, _z, 0)
    pltpu.sync_copy(zbuf_v, acc.at[pl.ds(s * ROWS_PER_TILE, ROWS_PER_TILE)])
    plsc.subcore_barrier()

    def _edge_chunk(j, carry):
        pltpu.sync_copy(ones_v, acc.at[ed_v.at[j, 1]], add=True)
        return carry

    lax.fori_loop(0, NCHUNK, _edge_chunk, 0)
    plsc.subcore_barrier()
    pltpu.sync_copy(acc.at[pl.ds(s * ROWS_PER_TILE, ROWS_PER_TILE)],
                    out_hbm.at[c, pl.ds(s * ROWS_PER_TILE, ROWS_PER_TILE)])


def _make_spmm():
    mesh = plsc.VectorSubcoreMesh(core_axis_name="c", subcore_axis_name="s")
    return pl.kernel(
        _spmm_body,
        out_type=jax.ShapeDtypeStruct((NC, N_PAD, HID), _F32),
        mesh=mesh,
        scratch_types=[
            pltpu.VMEM((NCHUNK, 2, CHUNK), jnp.int32),
            pltpu.VMEM((NBUF, CHUNK, HID), _F32),
            pltpu.VMEM((CHUNK, HID), _F32),
            pltpu.VMEM_SHARED((N_PAD, HID), _F32),
            [pltpu.SemaphoreType.DMA] * NBUF,
        ],
        compiler_params=pltpu.CompilerParams(use_tc_tiling_on_sc=False),
        name="gcn_spmm_sc",
    )


def _make_deg():
    mesh = plsc.VectorSubcoreMesh(core_axis_name="c", subcore_axis_name="s")
    return pl.kernel(
        _deg_body,
        out_type=jax.ShapeDtypeStruct((NC, N_PAD), _F32),
        mesh=mesh,
        scratch_types=[
            pltpu.VMEM((NCHUNK, 2, CHUNK), jnp.int32),
            pltpu.VMEM((CHUNK,), _F32),
            pltpu.VMEM((ROWS_PER_TILE,), _F32),
            pltpu.VMEM_SHARED((N_PAD,), _F32),
            pltpu.SemaphoreType.DMA,
        ],
        compiler_params=pltpu.CompilerParams(use_tc_tiling_on_sc=False),
        name="gcn_deg_sc",
    )


_BLK = 640


def _tc0_body(x_ref, w_ref, dinv_ref, h_ref):
    h_ref[...] = dinv_ref[...] * jnp.dot(x_ref[...], w_ref[...],
                                         preferred_element_type=_F32)


def _tc_mid_body(agg_ref, h_ref, dinv_ref, w_ref, b_ref, out_ref):
    a = agg_ref[0] + agg_ref[1] + h_ref[...]
    xn = jnp.maximum(dinv_ref[...] * a + b_ref[...], 0.0)
    out_ref[...] = dinv_ref[...] * jnp.dot(xn, w_ref[...],
                                           preferred_element_type=_F32)


def _tc_mid_nomm_body(agg_ref, h_ref, dinv_ref, b_ref, out_ref):
    a = agg_ref[0] + agg_ref[1] + h_ref[...]
    out_ref[...] = dinv_ref[...] * jnp.maximum(
        dinv_ref[...] * a + b_ref[...], 0.0)


def _tc_last_body(agg_ref, h_ref, dinv_ref, w_ref, b_ref, out_ref):
    a = dinv_ref[...] * (agg_ref[0] + agg_ref[1] + h_ref[...])
    out_ref[...] = jnp.dot(a, w_ref[...], preferred_element_type=_F32) + b_ref[...]


def _row_spec(width=128):
    return pl.BlockSpec((_BLK, width), lambda i: (i, 0))


def _full_spec(shape):
    nd = len(shape)
    return pl.BlockSpec(shape, lambda i: (0,) * nd)


def _tc0(x4, W1bd, dinv_rep):
    return pl.pallas_call(
        _tc0_body,
        grid=(NFLAT // _BLK,),
        in_specs=[
            pl.BlockSpec((_BLK, 4 * IN_DIM), lambda i: (i, 0)),
            _full_spec((4 * IN_DIM, 128)),
            _row_spec(),
        ],
        out_specs=_row_spec(),
        out_shape=jax.ShapeDtypeStruct((NFLAT, 128), _F32),
        name="gcn_tc0",
    )(x4, W1bd, dinv_rep)


def _tc_mid(agg, h, dinv_rep, Wbd, b_rep):
    return pl.pallas_call(
        _tc_mid_body,
        grid=(NFLAT // _BLK,),
        in_specs=[
            pl.BlockSpec((NC, _BLK, 128), lambda i: (0, i, 0)),
            _row_spec(), _row_spec(),
            _full_spec((128, 128)),
            _full_spec((1, 128)),
        ],
        out_specs=_row_spec(),
        out_shape=jax.ShapeDtypeStruct((NFLAT, 128), _F32),
        name="gcn_tc_mid",
    )(agg, h, dinv_rep, Wbd, b_rep)


def _tc_mid_nomm(agg, h, dinv_rep, b_rep):
    return pl.pallas_call(
        _tc_mid_nomm_body,
        grid=(NFLAT // _BLK,),
        in_specs=[
            pl.BlockSpec((NC, _BLK, 128), lambda i: (0, i, 0)),
            _row_spec(), _row_spec(),
            _full_spec((1, 128)),
        ],
        out_specs=_row_spec(),
        out_shape=jax.ShapeDtypeStruct((NFLAT, 128), _F32),
        name="gcn_tc_mid2",
    )(agg, h, dinv_rep, b_rep)


def _tc_last(agg, h, dinv_rep, W4bd, b4_rep):
    return pl.pallas_call(
        _tc_last_body,
        grid=(NFLAT // _BLK,),
        in_specs=[
            pl.BlockSpec((NC, _BLK, 128), lambda i: (0, i, 0)),
            _row_spec(), _row_spec(),
            _full_spec((128, 4 * OUT_DIM)),
            _full_spec((1, 4 * OUT_DIM)),
        ],
        out_specs=pl.BlockSpec((_BLK, 4 * OUT_DIM), lambda i: (i, 0)),
        out_shape=jax.ShapeDtypeStruct((NFLAT, 4 * OUT_DIM), _F32),
        name="gcn_tc_last",
    )(agg, h, dinv_rep, W4bd, b4_rep)


def _kron4(W):
    return jnp.kron(jnp.eye(4, dtype=_F32), W)


def kernel(x, edge_index, W1, b1, W2, b2, W3, b3, W4, b4):
    ed = (edge_index.astype(jnp.int32)
          .reshape(2, NCHUNK_REAL, CHUNK).transpose(1, 0, 2))
    seq = jnp.arange(NPADCH * CHUNK, dtype=jnp.int32)
    ped = jnp.stack([(seq % N).reshape(NPADCH, CHUNK),
                     (N + seq % (N_PAD - N)).reshape(NPADCH, CHUNK)], axis=1)

    x4 = jnp.pad(x, ((0, N_PAD - N), (0, 0))).reshape(NFLAT, 4 * IN_DIM)

    spmm = _make_spmm()
    degp = _make_deg()(ed, ped)
    dinv = lax.rsqrt(degp[0] + degp[1] + 1.0)
    dinv_rep = jnp.repeat(dinv, HID).reshape(NFLAT, 128)

    def flat(a):
        return a.reshape(NC, NFLAT, 128)

    h1 = _tc0(x4, _kron4(W1), dinv_rep)
    agg1 = flat(spmm(h1.reshape(N_PAD, HID), ed, ped))
    h2 = _tc_mid(agg1, h1, dinv_rep, _kron4(W2), jnp.tile(b1, 4)[None, :])
    agg2 = flat(spmm(h2.reshape(N_PAD, HID), ed, ped))
    h3 = _tc_mid(agg2, h2, dinv_rep, _kron4(W3), jnp.tile(b2, 4)[None, :])
    agg3 = flat(spmm(h3.reshape(N_PAD, HID), ed, ped))
    x4p = _tc_mid_nomm(agg3, h3, dinv_rep, jnp.tile(b3, 4)[None, :])
    agg4 = flat(spmm(x4p.reshape(N_PAD, HID), ed, ped))
    out = _tc_last(agg4, x4p, dinv_rep, _kron4(W4), jnp.tile(b4, 4)[None, :])
    return out.reshape(N_PAD, OUT_DIM)[:N]
